# Initial kernel scaffold; baseline (speedup 1.0000x reference)
import jax
import jax.numpy as jnp
from jax.experimental import pallas as pl


def kernel(x, tables, W1, b1, W2, b2, W3, b3):
    raise NotImplementedError("write your pallas kernel here")


# speedup vs baseline: 1.4375x; 1.4375x over previous
import functools

import jax
import jax.numpy as jnp
from jax import lax
from jax.experimental import pallas as pl
from jax.experimental.pallas import tpu as pltpu
from jax.experimental.pallas import tpu_sc as plsc

_VOCAB = 100000
_B = 16384
_NT = 14
_D = 5
_NC = 2
_NS = 16
_NW = _NC * _NS
_BPW = _B // _NW
_GCH = 128
_NG = _BPW // _GCH
_WSH = 3
_WW = 1 << _WSH
_NWIN = _NT * _VOCAB * _D // _WW


def _sc_gather_body(x_hbm, tables_hbm, out_hbm,
                    x_v, r0_v, r1_v, p_v, d3_v, rows_v, sem):
    wid = lax.axis_index("s") * _NC + lax.axis_index("c")
    base = wid * _BPW
    pltpu.sync_copy(x_hbm.at[pl.ds(base * 16, _BPW * 16)], x_v)

    def per_table(c, carry):
        col = c + 2
        word_off = c * (_VOCAB * _D)

        def per_group(g, carry2):
            def per_vec(o, carry3):
                pos = (lax.iota(jnp.int32, 16)
                       + (g * _GCH + o * 16)) * 16 + col
                vals = plsc.load_gather(x_v, [pos])
                w = vals.astype(jnp.int32) * _D + word_off
                r0 = lax.shift_right_logical(w, _WSH)
                r1 = jnp.minimum(r0 + 1, _NWIN - 1)
                r0_v[pl.ds(o * 16, 16)] = r0
                r1_v[pl.ds(o * 16, 16)] = r1
                p_v[pl.ds(o * 16, 16)] = jnp.bitwise_and(w, _WW - 1)
                return carry3
            lax.fori_loop(0, _GCH // 16, per_vec, carry2)

            cp0 = pltpu.async_copy(tables_hbm.at[r0_v], d3_v.at[0], sem)
            cp1 = pltpu.async_copy(tables_hbm.at[r1_v], d3_v.at[1], sem)
            cp0.wait()
            cp1.wait()

            for j in range(_D):
                jv = jnp.zeros((16,), jnp.int32) + j

                def per_vec2(o, carry3, jv=jv, j=j):
                    s_v = lax.iota(jnp.int32, 16) + o * 16
                    w = p_v[pl.ds(o * 16, 16)] + j
                    sel = lax.shift_right_logical(w, _WSH)
                    win = jnp.bitwise_and(w, _WW - 1)
                    vals = plsc.load_gather(d3_v, [sel, s_v, win])
                    plsc.store_scatter(rows_v, [s_v, jv], vals)
                    return carry3
                lax.fori_loop(0, _GCH // 16, per_vec2, 0)

            obase = c * _B + base + g * _GCH
            pltpu.sync_copy(rows_v, out_hbm.at[pl.ds(obase, _GCH), :])
            return carry2
        return lax.fori_loop(0, _NG, per_group, carry)

    lax.fori_loop(0, _NT, per_table, 0)


def _sc_gather(x2d, tables_win):
    mesh = plsc.VectorSubcoreMesh(core_axis_name="c", subcore_axis_name="s")
    fn = pl.kernel(
        _sc_gather_body,
        out_type=jax.ShapeDtypeStruct((_NT * _B, _D), jnp.float32),
        mesh=mesh,
        compiler_params=pltpu.CompilerParams(
            needs_layout_passes=False, use_tc_tiling_on_sc=False),
        scratch_types=[
            pltpu.VMEM((_BPW * 16,), jnp.float32),
            pltpu.VMEM((_GCH,), jnp.int32),
            pltpu.VMEM((_GCH,), jnp.int32),
            pltpu.VMEM((_GCH,), jnp.int32),
            pltpu.VMEM((2, _GCH, _WW), jnp.float32),
            pltpu.VMEM((_GCH, _D), jnp.float32),
            pltpu.SemaphoreType.DMA,
        ],
    )
    return fn(x2d, tables_win)


def _mlp_body(x_ref, emb_ref, w1_ref, b1_ref, w2_ref, b2_ref, w3_ref, b3_ref,
              out_ref):
    xb = x_ref[0]
    h = jnp.dot(xb[:, 0:2], w1_ref[0:2, :], preferred_element_type=jnp.float32)
    for c in range(_NT):
        h = h + jnp.dot(emb_ref[c], w1_ref[2 + 5 * c:7 + 5 * c, :],
                        preferred_element_type=jnp.float32)
    h = jnp.maximum(h + b1_ref[...], 0.0)
    h = jnp.maximum(
        jnp.dot(h, w2_ref[...], preferred_element_type=jnp.float32)
        + b2_ref[...], 0.0)
    z = (jnp.dot(h, w3_ref[...], preferred_element_type=jnp.float32)
         + b3_ref[...])
    out_ref[...] = jax.nn.sigmoid(z)


def _mlp(x, emb, W1, b1, W2, b2, W3, b3):
    bt = 2048
    grid = (_B // bt,)
    return pl.pallas_call(
        _mlp_body,
        grid=grid,
        in_specs=[
            pl.BlockSpec((1, bt, 16), lambda i: (0, i, 0)),
            pl.BlockSpec((_NT, bt, _D), lambda i: (0, i, 0)),
            pl.BlockSpec((72, 20), lambda i: (0, 0)),
            pl.BlockSpec((20,), lambda i: (0,)),
            pl.BlockSpec((20, 10), lambda i: (0, 0)),
            pl.BlockSpec((10,), lambda i: (0,)),
            pl.BlockSpec((10, 1), lambda i: (0, 0)),
            pl.BlockSpec((1,), lambda i: (0,)),
        ],
        out_specs=pl.BlockSpec((bt, 1), lambda i: (i, 0)),
        out_shape=jax.ShapeDtypeStruct((_B, 1), jnp.float32),
    )(x, emb, W1, b1, W2, b2, W3, b3)


def kernel(x, tables, W1, b1, W2, b2, W3, b3):
    x2d = x.reshape(_B * 16)
    tables_win = tables.reshape(_NWIN, _WW)
    emb = _sc_gather(x2d, tables_win).reshape(_NT, _B, _D)
    out = _mlp(x, emb, W1, b1, W2, b2, W3, b3)
    return out.reshape(_B)


# speedup vs baseline: 2.3452x; 1.6314x over previous
import functools

import jax
import jax.numpy as jnp
from jax import lax
from jax.experimental import pallas as pl
from jax.experimental.pallas import tpu as pltpu
from jax.experimental.pallas import tpu_sc as plsc

_VOCAB = 100000
_B = 16384
_NT = 14
_D = 5
_NC = 2
_NS = 16
_NW = _NC * _NS
_BPW = _B // _NW
_GCH = 128
_NG = _BPW // _GCH
_WSH = 3
_WW = 1 << _WSH
_PLANE = _NT * _VOCAB
_PROWS = _PLANE // _WW


def _sc_gather_body(x_hbm, tables_hbm, out_hbm,
                    x_v, r_v, p_v, d_v, rows_v, sem):
    wid = lax.axis_index("s") * _NC + lax.axis_index("c")
    base = wid * _BPW
    pltpu.sync_copy(x_hbm.at[pl.ds(base * 16, _BPW * 16)], x_v)

    def per_table(c, carry):
        col = c + 2
        word_off = c * _VOCAB

        def per_group(g, carry2):
            def per_vec(o, carry3):
                pos = (lax.iota(jnp.int32, 16)
                       + (g * _GCH + o * 16)) * 16 + col
                vals = plsc.load_gather(x_v, [pos])
                w = vals.astype(jnp.int32) + word_off
                r0 = lax.shift_right_logical(w, _WSH)
                for j in range(_D):
                    r_v[j, pl.ds(o * 16, 16)] = r0 + j * _PROWS
                p_v[pl.ds(o * 16, 16)] = jnp.bitwise_and(w, _WW - 1)
                return carry3
            lax.fori_loop(0, _GCH // 16, per_vec, carry2)

            cps = [
                pltpu.async_copy(tables_hbm.at[r_v.at[j]], d_v.at[j], sem)
                for j in range(_D)
            ]
            for cp in cps:
                cp.wait()

            for j in range(_D):
                jv = jnp.zeros((16,), jnp.int32) + j

                def per_vec2(o, carry3, jv=jv):
                    s_v = lax.iota(jnp.int32, 16) + o * 16
                    pv = p_v[pl.ds(o * 16, 16)]
                    vals = plsc.load_gather(d_v, [jv, s_v, pv])
                    plsc.store_scatter(rows_v, [s_v, jv], vals)
                    return carry3
                lax.fori_loop(0, _GCH // 16, per_vec2, 0)

            obase = c * _B + base + g * _GCH
            pltpu.sync_copy(rows_v, out_hbm.at[pl.ds(obase, _GCH), :])
            return carry2
        return lax.fori_loop(0, _NG, per_group, carry)

    lax.fori_loop(0, _NT, per_table, 0)


def _sc_gather(x2d, tables_win):
    mesh = plsc.VectorSubcoreMesh(core_axis_name="c", subcore_axis_name="s")
    fn = pl.kernel(
        _sc_gather_body,
        out_type=jax.ShapeDtypeStruct((_NT * _B, _D), jnp.float32),
        mesh=mesh,
        compiler_params=pltpu.CompilerParams(
            needs_layout_passes=False, use_tc_tiling_on_sc=False),
        scratch_types=[
            pltpu.VMEM((_BPW * 16,), jnp.float32),
            pltpu.VMEM((_D, _GCH), jnp.int32),
            pltpu.VMEM((_GCH,), jnp.int32),
            pltpu.VMEM((_D, _GCH, _WW), jnp.float32),
            pltpu.VMEM((_GCH, _D), jnp.float32),
            pltpu.SemaphoreType.DMA,
        ],
    )
    return fn(x2d, tables_win)


def _mlp_body(x_ref, emb_ref, w1_ref, b1_ref, w2_ref, b2_ref, w3_ref, b3_ref,
              out_ref):
    xb = x_ref[0]
    h = jnp.dot(xb[:, 0:2], w1_ref[0:2, :], preferred_element_type=jnp.float32)
    for c in range(_NT):
        h = h + jnp.dot(emb_ref[c], w1_ref[2 + 5 * c:7 + 5 * c, :],
                        preferred_element_type=jnp.float32)
    h = jnp.maximum(h + b1_ref[...], 0.0)
    h = jnp.maximum(
        jnp.dot(h, w2_ref[...], preferred_element_type=jnp.float32)
        + b2_ref[...], 0.0)
    z = (jnp.dot(h, w3_ref[...], preferred_element_type=jnp.float32)
         + b3_ref[...])
    out_ref[...] = jax.nn.sigmoid(z)


def _mlp(x, emb, W1, b1, W2, b2, W3, b3):
    bt = 2048
    grid = (_B // bt,)
    return pl.pallas_call(
        _mlp_body,
        grid=grid,
        in_specs=[
            pl.BlockSpec((1, bt, 16), lambda i: (0, i, 0)),
            pl.BlockSpec((_NT, bt, _D), lambda i: (0, i, 0)),
            pl.BlockSpec((72, 20), lambda i: (0, 0)),
            pl.BlockSpec((20,), lambda i: (0,)),
            pl.BlockSpec((20, 10), lambda i: (0, 0)),
            pl.BlockSpec((10,), lambda i: (0,)),
            pl.BlockSpec((10, 1), lambda i: (0, 0)),
            pl.BlockSpec((1,), lambda i: (0,)),
        ],
        out_specs=pl.BlockSpec((bt, 1), lambda i: (i, 0)),
        out_shape=jax.ShapeDtypeStruct((_B, 1), jnp.float32),
    )(x, emb, W1, b1, W2, b2, W3, b3)


def kernel(x, tables, W1, b1, W2, b2, W3, b3):
    x2d = x.reshape(_B * 16)
    tables_win = tables.transpose(2, 0, 1).reshape(_D * _PROWS, _WW)
    emb = _sc_gather(x2d, tables_win).reshape(_NT, _B, _D)
    out = _mlp(x, emb, W1, b1, W2, b2, W3, b3)
    return out.reshape(_B)


# speedup vs baseline: 2.6135x; 1.1144x over previous
import functools

import jax
import jax.numpy as jnp
from jax import lax
from jax.experimental import pallas as pl
from jax.experimental.pallas import tpu as pltpu
from jax.experimental.pallas import tpu_sc as plsc

_VOCAB = 100000
_B = 16384
_NT = 14
_D = 5
_NC = 2
_NS = 16
_NW = _NC * _NS
_BPW = _B // _NW
_GCH = 128
_NG = _BPW // _GCH
_WSH = 3
_WW = 1 << _WSH
_PLANE = _NT * _VOCAB
_PROWS = _PLANE // _WW


_NCH = _NT * _NG


def _sc_gather_body(x_hbm, tables_hbm, out_hbm, x_v,
                    r0_v, p0_v, d0_v, r1_v, p1_v, d1_v, rows_v, sem0, sem1):
    wid = lax.axis_index("s") * _NC + lax.axis_index("c")
    base = wid * _BPW
    pltpu.sync_copy(x_hbm.at[pl.ds(base * 16, _BPW * 16)], x_v)

    def build_fire(t, r_v, p_v, d_v, sem):
        c = lax.shift_right_logical(t, 2)
        g = jnp.bitwise_and(t, _NG - 1)
        col = c + 2
        word_off = c * _VOCAB

        def per_vec(o, carry):
            pos = (lax.iota(jnp.int32, 16) + (g * _GCH + o * 16)) * 16 + col
            vals = plsc.load_gather(x_v, [pos])
            w = vals.astype(jnp.int32) + word_off
            r0 = lax.shift_right_logical(w, _WSH)
            for j in range(_D):
                r_v[j, pl.ds(o * 16, 16)] = r0 + j * _PROWS
            p_v[pl.ds(o * 16, 16)] = jnp.bitwise_and(w, _WW - 1)
            return carry
        lax.fori_loop(0, _GCH // 16, per_vec, 0)
        for j in range(_D):
            pltpu.async_copy(tables_hbm.at[r_v.at[j]], d_v.at[j], sem)

    def drain_extract(t, r_v, p_v, d_v, sem):
        for j in range(_D):
            pltpu.make_async_copy(
                tables_hbm.at[r_v.at[j]], d_v.at[j], sem).wait()

        def per_vec(o, carry):
            s_v = lax.iota(jnp.int32, 16) + o * 16
            pv = p_v[pl.ds(o * 16, 16)]
            for j in range(_D):
                jv = jnp.zeros((16,), jnp.int32) + j
                vals = plsc.load_gather(d_v, [jv, s_v, pv])
                plsc.store_scatter(rows_v, [s_v, jv], vals)
            return carry
        lax.fori_loop(0, _GCH // 16, per_vec, 0)

        c = lax.shift_right_logical(t, 2)
        g = jnp.bitwise_and(t, _NG - 1)
        obase = c * _B + base + g * _GCH
        pltpu.sync_copy(rows_v, out_hbm.at[pl.ds(obase, _GCH), :])

    build_fire(jnp.int32(0), r0_v, p0_v, d0_v, sem0)

    def body(t2, carry):
        te = t2 * 2
        build_fire(te + 1, r1_v, p1_v, d1_v, sem1)
        drain_extract(te, r0_v, p0_v, d0_v, sem0)

        @pl.when(t2 != _NCH // 2 - 1)
        def _():
            build_fire(te + 2, r0_v, p0_v, d0_v, sem0)
        drain_extract(te + 1, r1_v, p1_v, d1_v, sem1)
        return carry

    lax.fori_loop(0, _NCH // 2, body, 0)


def _sc_gather(x2d, tables_win):
    mesh = plsc.VectorSubcoreMesh(core_axis_name="c", subcore_axis_name="s")
    fn = pl.kernel(
        _sc_gather_body,
        out_type=jax.ShapeDtypeStruct((_NT * _B, _D), jnp.float32),
        mesh=mesh,
        compiler_params=pltpu.CompilerParams(
            needs_layout_passes=False, use_tc_tiling_on_sc=False),
        scratch_types=[
            pltpu.VMEM((_BPW * 16,), jnp.float32),
            pltpu.VMEM((_D, _GCH), jnp.int32),
            pltpu.VMEM((_GCH,), jnp.int32),
            pltpu.VMEM((_D, _GCH, _WW), jnp.float32),
            pltpu.VMEM((_D, _GCH), jnp.int32),
            pltpu.VMEM((_GCH,), jnp.int32),
            pltpu.VMEM((_D, _GCH, _WW), jnp.float32),
            pltpu.VMEM((_GCH, _D), jnp.float32),
            pltpu.SemaphoreType.DMA,
            pltpu.SemaphoreType.DMA,
        ],
    )
    return fn(x2d, tables_win)


def _mlp_body(x_ref, emb_ref, w1_ref, b1_ref, w2_ref, b2_ref, w3_ref, b3_ref,
              out_ref):
    xb = x_ref[0]
    h = jnp.dot(xb[:, 0:2], w1_ref[0:2, :], preferred_element_type=jnp.float32)
    for c in range(_NT):
        h = h + jnp.dot(emb_ref[c], w1_ref[2 + 5 * c:7 + 5 * c, :],
                        preferred_element_type=jnp.float32)
    h = jnp.maximum(h + b1_ref[...], 0.0)
    h = jnp.maximum(
        jnp.dot(h, w2_ref[...], preferred_element_type=jnp.float32)
        + b2_ref[...], 0.0)
    z = (jnp.dot(h, w3_ref[...], preferred_element_type=jnp.float32)
         + b3_ref[...])
    out_ref[...] = jax.nn.sigmoid(z)


def _mlp(x, emb, W1, b1, W2, b2, W3, b3):
    bt = 2048
    grid = (_B // bt,)
    return pl.pallas_call(
        _mlp_body,
        grid=grid,
        in_specs=[
            pl.BlockSpec((1, bt, 16), lambda i: (0, i, 0)),
            pl.BlockSpec((_NT, bt, _D), lambda i: (0, i, 0)),
            pl.BlockSpec((72, 20), lambda i: (0, 0)),
            pl.BlockSpec((20,), lambda i: (0,)),
            pl.BlockSpec((20, 10), lambda i: (0, 0)),
            pl.BlockSpec((10,), lambda i: (0,)),
            pl.BlockSpec((10, 1), lambda i: (0, 0)),
            pl.BlockSpec((1,), lambda i: (0,)),
        ],
        out_specs=pl.BlockSpec((bt, 1), lambda i: (i, 0)),
        out_shape=jax.ShapeDtypeStruct((_B, 1), jnp.float32),
    )(x, emb, W1, b1, W2, b2, W3, b3)


def kernel(x, tables, W1, b1, W2, b2, W3, b3):
    x2d = x.reshape(_B * 16)
    tables_win = tables.transpose(2, 0, 1).reshape(_D * _PROWS, _WW)
    emb = _sc_gather(x2d, tables_win).reshape(_NT, _B, _D)
    out = _mlp(x, emb, W1, b1, W2, b2, W3, b3)
    return out.reshape(_B)


# speedup vs baseline: 3.2208x; 1.2324x over previous
import functools

import jax
import jax.numpy as jnp
from jax import lax
from jax.experimental import pallas as pl
from jax.experimental.pallas import tpu as pltpu
from jax.experimental.pallas import tpu_sc as plsc

_VOCAB = 100000
_B = 16384
_NT = 14
_D = 5
_NC = 2
_NS = 16
_NW = _NC * _NS
_BPW = _B // _NW
_GCH = 128
_NG = _BPW // _GCH
_WSH = 3
_WW = 1 << _WSH
_PLANE = _NT * _VOCAB
_PROWS = _PLANE // _WW


_NCH = _NT * _NG


def _sc_gather_body(x_hbm, tables_hbm, out_hbm, x_v,
                    r0_v, p0_v, d0_v, r1_v, p1_v, d1_v, rows_v, sem0, sem1):
    wid = lax.axis_index("s") * _NC + lax.axis_index("c")
    base = wid * _BPW
    pltpu.sync_copy(x_hbm.at[pl.ds(base * 16, _BPW * 16)], x_v)

    def build_fire(t, r_v, p_v, d_v, sem):
        c = lax.shift_right_logical(t, 2)
        g = jnp.bitwise_and(t, _NG - 1)
        col = c + 2
        word_off = c * _VOCAB

        def per_vec(o, carry):
            pos = (lax.iota(jnp.int32, 16) + (g * _GCH + o * 16)) * 16 + col
            vals = plsc.load_gather(x_v, [pos])
            w = vals.astype(jnp.int32) + word_off
            r0 = lax.shift_right_logical(w, _WSH)
            for j in range(_D):
                r_v[j, pl.ds(o * 16, 16)] = r0 + j * _PROWS
            p_v[pl.ds(o * 16, 16)] = jnp.bitwise_and(w, _WW - 1)
            return carry
        lax.fori_loop(0, _GCH // 16, per_vec, 0)
        for j in range(_D):
            pltpu.async_copy(tables_hbm.at[r_v.at[j]], d_v.at[j], sem)

    def drain_extract(t, r_v, p_v, d_v, sem):
        for j in range(_D):
            pltpu.make_async_copy(
                tables_hbm.at[r_v.at[j]], d_v.at[j], sem).wait()

        def per_vec(o, carry):
            s_v = lax.iota(jnp.int32, 16) + o * 16
            pv = p_v[pl.ds(o * 16, 16)]
            for j in range(_D):
                jv = jnp.zeros((16,), jnp.int32) + j
                vals = plsc.load_gather(d_v, [jv, s_v, pv])
                rows_v[j, pl.ds(o * 16, 16)] = vals
            return carry
        lax.fori_loop(0, _GCH // 16, per_vec, 0)

        c = lax.shift_right_logical(t, 2)
        g = jnp.bitwise_and(t, _NG - 1)
        obase = base + g * _GCH
        pltpu.sync_copy(rows_v,
                        out_hbm.at[pl.ds(c * _D, _D), pl.ds(obase, _GCH)])

    build_fire(jnp.int32(0), r0_v, p0_v, d0_v, sem0)

    def body(t2, carry):
        te = t2 * 2
        build_fire(te + 1, r1_v, p1_v, d1_v, sem1)
        drain_extract(te, r0_v, p0_v, d0_v, sem0)

        @pl.when(t2 != _NCH // 2 - 1)
        def _():
            build_fire(te + 2, r0_v, p0_v, d0_v, sem0)
        drain_extract(te + 1, r1_v, p1_v, d1_v, sem1)
        return carry

    lax.fori_loop(0, _NCH // 2, body, 0)


def _sc_gather(x2d, tables_win):
    mesh = plsc.VectorSubcoreMesh(core_axis_name="c", subcore_axis_name="s")
    fn = pl.kernel(
        _sc_gather_body,
        out_type=jax.ShapeDtypeStruct((_NT * _D, _B), jnp.float32),
        mesh=mesh,
        compiler_params=pltpu.CompilerParams(
            needs_layout_passes=False, use_tc_tiling_on_sc=False),
        scratch_types=[
            pltpu.VMEM((_BPW * 16,), jnp.float32),
            pltpu.VMEM((_D, _GCH), jnp.int32),
            pltpu.VMEM((_GCH,), jnp.int32),
            pltpu.VMEM((_D, _GCH, _WW), jnp.float32),
            pltpu.VMEM((_D, _GCH), jnp.int32),
            pltpu.VMEM((_GCH,), jnp.int32),
            pltpu.VMEM((_D, _GCH, _WW), jnp.float32),
            pltpu.VMEM((_D, _GCH), jnp.float32),
            pltpu.SemaphoreType.DMA,
            pltpu.SemaphoreType.DMA,
        ],
    )
    return fn(x2d, tables_win)


def _mlp_body(x_ref, emb_ref, w1_ref, b1_ref, w2_ref, b2_ref, w3_ref, b3_ref,
              out_ref):
    xb = x_ref[0]
    h = jnp.dot(xb[:, 0:2], w1_ref[0:2, :], preferred_element_type=jnp.float32)
    h = h + jax.lax.dot_general(
        emb_ref[...], w1_ref[2:72, :], (((0,), (0,)), ((), ())),
        preferred_element_type=jnp.float32)
    h = jnp.maximum(h + b1_ref[...], 0.0)
    h = jnp.maximum(
        jnp.dot(h, w2_ref[...], preferred_element_type=jnp.float32)
        + b2_ref[...], 0.0)
    z = (jnp.dot(h, w3_ref[...], preferred_element_type=jnp.float32)
         + b3_ref[...])
    out_ref[...] = jax.nn.sigmoid(z)


def _mlp(x, emb, W1, b1, W2, b2, W3, b3):
    bt = 2048
    grid = (_B // bt,)
    return pl.pallas_call(
        _mlp_body,
        grid=grid,
        in_specs=[
            pl.BlockSpec((1, bt, 16), lambda i: (0, i, 0)),
            pl.BlockSpec((_NT * _D, bt), lambda i: (0, i)),
            pl.BlockSpec((72, 20), lambda i: (0, 0)),
            pl.BlockSpec((20,), lambda i: (0,)),
            pl.BlockSpec((20, 10), lambda i: (0, 0)),
            pl.BlockSpec((10,), lambda i: (0,)),
            pl.BlockSpec((10, 1), lambda i: (0, 0)),
            pl.BlockSpec((1,), lambda i: (0,)),
        ],
        out_specs=pl.BlockSpec((bt, 1), lambda i: (i, 0)),
        out_shape=jax.ShapeDtypeStruct((_B, 1), jnp.float32),
    )(x, emb, W1, b1, W2, b2, W3, b3)


def kernel(x, tables, W1, b1, W2, b2, W3, b3):
    x2d = x.reshape(_B * 16)
    tables_win = tables.transpose(2, 0, 1).reshape(_D * _PROWS, _WW)
    emb = _sc_gather(x2d, tables_win)
    out = _mlp(x, emb, W1, b1, W2, b2, W3, b3)
    return out.reshape(_B)


# speedup vs baseline: 3.2284x; 1.0024x over previous
import jax
import jax.numpy as jnp
from jax import lax
from jax.experimental import pallas as pl
from jax.experimental.pallas import tpu as pltpu
from jax.experimental.pallas import tpu_sc as plsc

_VOCAB = 100000
_B = 16384
_NT = 14
_D = 5
_NC = 2
_NS = 16
_NW = _NC * _NS
_BPW = _B // _NW
_GCH = 128
_NG = _BPW // _GCH
_WSH = 3
_WW = 1 << _WSH
_PLANE = _NT * _VOCAB
_PROWS = _PLANE // _WW


_NCH = _NT * _NG


def _sc_gather_body(x_hbm, tables_hbm, out_hbm, x_v,
                    r0_v, p0_v, d0_v, r1_v, p1_v, d1_v, rows_v, sem0, sem1):
    wid = lax.axis_index("s") * _NC + lax.axis_index("c")
    base = wid * _BPW
    pltpu.sync_copy(x_hbm.at[pl.ds(base * 16, _BPW * 16)], x_v)

    def build_fire(t, r_v, p_v, d_v, sem):
        c = lax.shift_right_logical(t, 2)
        g = jnp.bitwise_and(t, _NG - 1)
        col = c + 2
        word_off = c * _VOCAB

        def per_vec(o, carry):
            pos = (lax.iota(jnp.int32, 16) + (g * _GCH + o * 16)) * 16 + col
            vals = plsc.load_gather(x_v, [pos])
            w = vals.astype(jnp.int32) + word_off
            r0 = lax.shift_right_logical(w, _WSH)
            for j in range(_D):
                r_v[j, pl.ds(o * 16, 16)] = r0 + j * _PROWS
            p_v[pl.ds(o * 16, 16)] = jnp.bitwise_and(w, _WW - 1)
            return carry
        lax.fori_loop(0, _GCH // 16, per_vec, 0)
        for j in range(_D):
            pltpu.async_copy(tables_hbm.at[r_v.at[j]], d_v.at[j], sem)

    def drain_extract(t, r_v, p_v, d_v, sem):
        for j in range(_D):
            pltpu.make_async_copy(
                tables_hbm.at[r_v.at[j]], d_v.at[j], sem).wait()

        def per_vec(o, carry):
            s_v = lax.iota(jnp.int32, 16) + o * 16
            pv = p_v[pl.ds(o * 16, 16)]
            for j in range(_D):
                jv = jnp.zeros((16,), jnp.int32) + j
                vals = plsc.load_gather(d_v, [jv, s_v, pv])
                rows_v[j, pl.ds(o * 16, 16)] = vals
            return carry
        lax.fori_loop(0, _GCH // 16, per_vec, 0)

        c = lax.shift_right_logical(t, 2)
        g = jnp.bitwise_and(t, _NG - 1)
        obase = base + g * _GCH
        pltpu.sync_copy(rows_v,
                        out_hbm.at[pl.ds(c * _D, _D), pl.ds(obase, _GCH)])

    build_fire(jnp.int32(0), r0_v, p0_v, d0_v, sem0)

    def body(t2, carry):
        te = t2 * 2
        build_fire(te + 1, r1_v, p1_v, d1_v, sem1)
        drain_extract(te, r0_v, p0_v, d0_v, sem0)

        @pl.when(t2 != _NCH // 2 - 1)
        def _():
            build_fire(te + 2, r0_v, p0_v, d0_v, sem0)
        drain_extract(te + 1, r1_v, p1_v, d1_v, sem1)
        return carry

    lax.fori_loop(0, _NCH // 2, body, 0)


def _sc_gather(x2d, tables_win):
    mesh = plsc.VectorSubcoreMesh(core_axis_name="c", subcore_axis_name="s")
    fn = pl.kernel(
        _sc_gather_body,
        out_type=jax.ShapeDtypeStruct((_NT * _D, _B), jnp.float32),
        mesh=mesh,
        compiler_params=pltpu.CompilerParams(
            needs_layout_passes=False, use_tc_tiling_on_sc=False),
        scratch_types=[
            pltpu.VMEM((_BPW * 16,), jnp.float32),
            pltpu.VMEM((_D, _GCH), jnp.int32),
            pltpu.VMEM((_GCH,), jnp.int32),
            pltpu.VMEM((_D, _GCH, _WW), jnp.float32),
            pltpu.VMEM((_D, _GCH), jnp.int32),
            pltpu.VMEM((_GCH,), jnp.int32),
            pltpu.VMEM((_D, _GCH, _WW), jnp.float32),
            pltpu.VMEM((_D, _GCH), jnp.float32),
            pltpu.SemaphoreType.DMA,
            pltpu.SemaphoreType.DMA,
        ],
    )
    return fn(x2d, tables_win)


def _mlp_body(x_ref, emb_ref, w1_ref, b1_ref, w2_ref, b2_ref, w3_ref, b3_ref,
              out_ref):
    xb = x_ref[0]
    h = jnp.dot(xb[:, 0:2], w1_ref[0:2, :], preferred_element_type=jnp.float32)
    h = h + jax.lax.dot_general(
        emb_ref[...], w1_ref[2:72, :], (((0,), (0,)), ((), ())),
        preferred_element_type=jnp.float32)
    h = jnp.maximum(h + b1_ref[...], 0.0)
    h = jnp.maximum(
        jnp.dot(h, w2_ref[...], preferred_element_type=jnp.float32)
        + b2_ref[...], 0.0)
    z = (jnp.dot(h, w3_ref[...], preferred_element_type=jnp.float32)
         + b3_ref[...])
    out_ref[...] = jax.nn.sigmoid(z)


def _mlp(x, emb, W1, b1, W2, b2, W3, b3):
    bt = 2048
    grid = (_B // bt,)
    return pl.pallas_call(
        _mlp_body,
        grid=grid,
        in_specs=[
            pl.BlockSpec((1, bt, 16), lambda i: (0, i, 0)),
            pl.BlockSpec((_NT * _D, bt), lambda i: (0, i)),
            pl.BlockSpec((72, 20), lambda i: (0, 0)),
            pl.BlockSpec((20,), lambda i: (0,)),
            pl.BlockSpec((20, 10), lambda i: (0, 0)),
            pl.BlockSpec((10,), lambda i: (0,)),
            pl.BlockSpec((10, 1), lambda i: (0, 0)),
            pl.BlockSpec((1,), lambda i: (0,)),
        ],
        out_specs=pl.BlockSpec((bt, 1), lambda i: (i, 0)),
        out_shape=jax.ShapeDtypeStruct((_B, 1), jnp.float32),
    )(x, emb, W1, b1, W2, b2, W3, b3)


def kernel(x, tables, W1, b1, W2, b2, W3, b3):
    x2d = x.reshape(_B * 16)
    tables_win = tables.transpose(2, 0, 1).reshape(_D * _PROWS, _WW)
    emb = _sc_gather(x2d, tables_win)
    out = _mlp(x, emb, W1, b1, W2, b2, W3, b3)
    return out.reshape(_B)
